# baseline (speedup 1.0000x reference)
from kernel_sc import kernel_sc as kernel
